# baseline (speedup 1.0000x reference)
import functools

import jax
import jax.numpy as jnp
from jax import lax
from jax.experimental import pallas as pl
from jax.experimental.pallas import tpu as pltpu
from jax.experimental.pallas import tpu_sc as plsc

_N = 10000
_E = 320000
_C = 128
_H = 256

_NC = 2
_NS = 16
_B = 64
_EP = 327680
_NP = 10240
_RPS = _NP // _NS


def _sc_segment_sum(table, idxs, dsts, zeros, epc):
  eps = epc // _NS
  nb = eps // _B
  assert eps % _B == 0 and nb % 2 == 0

  mesh = plsc.VectorSubcoreMesh(core_axis_name="c", subcore_axis_name="s")

  @functools.partial(
      pl.kernel,
      out_type=jax.ShapeDtypeStruct((_NC, _NP, _C), jnp.float32),
      mesh=mesh,
      scratch_types=[
          pltpu.VMEM((_B,), jnp.int32),
          pltpu.VMEM((_B,), jnp.int32),
          pltpu.VMEM((_B, _C), jnp.float32),
          pltpu.VMEM_SHARED((_NP, _C), jnp.float32),
          pltpu.SemaphoreType.DMA,
      ],
  )
  def k(table_h, idxs_h, dsts_h, zero_h, out_h,
        idx0, dstb0, buf0, acc, sem0):
    c = lax.axis_index("c")
    s = lax.axis_index("s")

    pltpu.sync_copy(zero_h.at[pl.ds(s * _RPS, _RPS)],
                    acc.at[pl.ds(s * _RPS, _RPS)])
    plsc.subcore_barrier()

    base = c * epc + s * eps

    def body(i, carry):
      off = pl.multiple_of(base + i * _B, 8)
      pltpu.sync_copy(idxs_h.at[pl.ds(off, _B)], idx0)
      pltpu.async_copy(table_h.at[idx0], buf0, sem0).wait()
      pltpu.sync_copy(dsts_h.at[pl.ds(off, _B)], dstb0)
      pltpu.sync_copy(buf0, acc.at[dstb0], add=True)
      return carry

    lax.fori_loop(0, nb, body, 0)
    plsc.subcore_barrier()
    pltpu.sync_copy(acc.at[pl.ds(s * _RPS, _RPS)],
                    out_h.at[c, pl.ds(s * _RPS, _RPS)])

  return k(table, idxs, dsts, zeros)


_BLK = 400


def _mlp1_body(x_ref, p_ref, wa_ref, ba_ref, wb_ref, bb_ref, h_ref):
  t = x_ref[...] + p_ref[0] + p_ref[1]
  a = jnp.maximum(
      jnp.dot(t, wa_ref[...], preferred_element_type=jnp.float32)
      + ba_ref[...], 0.0)
  h = jnp.maximum(
      jnp.dot(a, wb_ref[...], preferred_element_type=jnp.float32)
      + bb_ref[...], 0.0)
  h_ref[...] = h


def _mlp1(x, p, W1a, b1a, W1b, b1b):
  return pl.pallas_call(
      _mlp1_body,
      grid=(_N // _BLK,),
      in_specs=[
          pl.BlockSpec((_BLK, _C), lambda i: (i, 0)),
          pl.BlockSpec((_NC, _BLK, _C), lambda i: (0, i, 0)),
          pl.BlockSpec((_C, _H), lambda i: (0, 0)),
          pl.BlockSpec((1, _H), lambda i: (0, 0)),
          pl.BlockSpec((_H, _H), lambda i: (0, 0)),
          pl.BlockSpec((1, _H), lambda i: (0, 0)),
      ],
      out_specs=pl.BlockSpec((_BLK, _H), lambda i: (i, 0)),
      out_shape=jax.ShapeDtypeStruct((_N, _H), jnp.float32),
  )(x, p, W1a, b1a.reshape(1, _H), W1b, b1b.reshape(1, _H))


def _mlp2_body(h_ref, p_ref, wa_ref, ba_ref, wb_ref, bb_ref, o_ref):
  t = h_ref[...] + jnp.concatenate([p_ref[0], p_ref[1]], axis=1)
  z = jnp.maximum(
      jnp.dot(t, wa_ref[...], preferred_element_type=jnp.float32)
      + ba_ref[...], 0.0)
  u = jnp.dot(z, wb_ref[...], preferred_element_type=jnp.float32) + bb_ref[...]
  o_ref[...] = 1.0 / (1.0 + jnp.exp(-u))


def _mlp2(h, p, W2a, b2a, W2b, b2b):
  return pl.pallas_call(
      _mlp2_body,
      grid=(_N // _BLK,),
      in_specs=[
          pl.BlockSpec((_BLK, _H), lambda i: (i, 0)),
          pl.BlockSpec((_NC, _BLK, _C), lambda i: (0, i, 0)),
          pl.BlockSpec((_H, _H), lambda i: (0, 0)),
          pl.BlockSpec((1, _H), lambda i: (0, 0)),
          pl.BlockSpec((_H, _C), lambda i: (0, 0)),
          pl.BlockSpec((1, _C), lambda i: (0, 0)),
      ],
      out_specs=pl.BlockSpec((_BLK, _C), lambda i: (i, 0)),
      out_shape=jax.ShapeDtypeStruct((_N, _C), jnp.float32),
  )(h, p, W2a, b2a.reshape(1, _H), W2b, b2b.reshape(1, _C))


def kernel(x, edge_index, W1a, b1a, W1b, b1b, W2a, b2a, W2b, b2b):
  src = edge_index[0].astype(jnp.int32)
  dst = edge_index[1].astype(jnp.int32)
  pad = _EP - _E
  srcp = jnp.concatenate([src, jnp.zeros((pad,), jnp.int32)])
  dstp = jnp.concatenate([dst, jnp.full((pad,), _N, jnp.int32)])
  zeros = jnp.zeros((_NP, _C), jnp.float32)

  idx2 = (srcp[None, :] * 2
          + jnp.arange(_NC, dtype=jnp.int32)[:, None]).reshape(_NC * _EP)
  dst2 = jnp.concatenate([dstp, dstp])

  p1 = _sc_segment_sum(x, srcp, dstp, zeros, _EP // _NC)[:, :_N]
  h = _mlp1(x, p1, W1a, b1a, W1b, b1b)
  p2 = _sc_segment_sum(h.reshape(2 * _N, _C), idx2, dst2, zeros,
                       _EP)[:, :_N]
  return _mlp2(h, p2, W2a, b2a, W2b, b2b)


# speedup vs baseline: 1.0734x; 1.0734x over previous
import functools

import jax
import jax.numpy as jnp
from jax import lax
from jax.experimental import pallas as pl
from jax.experimental.pallas import tpu as pltpu
from jax.experimental.pallas import tpu_sc as plsc

_N = 10000
_E = 320000
_C = 128
_H = 256

_NC = 2
_NS = 16
_B = 80
_EP = 327680
_NP = 10240
_RPS = _NP // _NS


def _sc_segment_sum(table, idxs, dsts, zeros, epc):
  eps = epc // _NS
  nb = eps // _B
  assert eps % _B == 0 and nb % 2 == 0

  mesh = plsc.VectorSubcoreMesh(core_axis_name="c", subcore_axis_name="s")

  @functools.partial(
      pl.kernel,
      out_type=jax.ShapeDtypeStruct((_NC, _NP, _C), jnp.float32),
      mesh=mesh,
      scratch_types=[
          pltpu.VMEM((_B,), jnp.int32),
          pltpu.VMEM((_B,), jnp.int32),
          pltpu.VMEM((_B, _C), jnp.float32),
          pltpu.VMEM_SHARED((_NP, _C), jnp.float32),
          pltpu.SemaphoreType.DMA,
      ],
  )
  def k(table_h, idxs_h, dsts_h, zero_h, out_h,
        idx0, dstb0, buf0, acc, sem0):
    c = lax.axis_index("c")
    s = lax.axis_index("s")

    pltpu.sync_copy(zero_h.at[pl.ds(s * _RPS, _RPS)],
                    acc.at[pl.ds(s * _RPS, _RPS)])
    plsc.subcore_barrier()

    base = c * epc + s * eps

    def body(i, carry):
      off = pl.multiple_of(base + i * _B, 8)
      pltpu.sync_copy(idxs_h.at[pl.ds(off, _B)], idx0)
      pltpu.async_copy(table_h.at[idx0], buf0, sem0).wait()
      pltpu.sync_copy(dsts_h.at[pl.ds(off, _B)], dstb0)
      pltpu.sync_copy(buf0, acc.at[dstb0], add=True)
      return carry

    lax.fori_loop(0, nb, body, 0)
    plsc.subcore_barrier()
    pltpu.sync_copy(acc.at[pl.ds(s * _RPS, _RPS)],
                    out_h.at[c, pl.ds(s * _RPS, _RPS)])

  return k(table, idxs, dsts, zeros)


_BLK = 400


def _mlp1_body(x_ref, p_ref, wa_ref, ba_ref, wb_ref, bb_ref, h_ref):
  t = x_ref[...] + p_ref[0] + p_ref[1]
  a = jnp.maximum(
      jnp.dot(t, wa_ref[...], preferred_element_type=jnp.float32)
      + ba_ref[...], 0.0)
  h = jnp.maximum(
      jnp.dot(a, wb_ref[...], preferred_element_type=jnp.float32)
      + bb_ref[...], 0.0)
  h_ref[...] = h


def _mlp1(x, p, W1a, b1a, W1b, b1b):
  return pl.pallas_call(
      _mlp1_body,
      grid=(_N // _BLK,),
      in_specs=[
          pl.BlockSpec((_BLK, _C), lambda i: (i, 0)),
          pl.BlockSpec((_NC, _BLK, _C), lambda i: (0, i, 0)),
          pl.BlockSpec((_C, _H), lambda i: (0, 0)),
          pl.BlockSpec((1, _H), lambda i: (0, 0)),
          pl.BlockSpec((_H, _H), lambda i: (0, 0)),
          pl.BlockSpec((1, _H), lambda i: (0, 0)),
      ],
      out_specs=pl.BlockSpec((_BLK, _H), lambda i: (i, 0)),
      out_shape=jax.ShapeDtypeStruct((_N, _H), jnp.float32),
  )(x, p, W1a, b1a.reshape(1, _H), W1b, b1b.reshape(1, _H))


def _mlp2_body(h_ref, p_ref, wa_ref, ba_ref, wb_ref, bb_ref, o_ref):
  t = h_ref[...] + jnp.concatenate([p_ref[0], p_ref[1]], axis=1)
  z = jnp.maximum(
      jnp.dot(t, wa_ref[...], preferred_element_type=jnp.float32)
      + ba_ref[...], 0.0)
  u = jnp.dot(z, wb_ref[...], preferred_element_type=jnp.float32) + bb_ref[...]
  o_ref[...] = 1.0 / (1.0 + jnp.exp(-u))


def _mlp2(h, p, W2a, b2a, W2b, b2b):
  return pl.pallas_call(
      _mlp2_body,
      grid=(_N // _BLK,),
      in_specs=[
          pl.BlockSpec((_BLK, _H), lambda i: (i, 0)),
          pl.BlockSpec((_NC, _BLK, _C), lambda i: (0, i, 0)),
          pl.BlockSpec((_H, _H), lambda i: (0, 0)),
          pl.BlockSpec((1, _H), lambda i: (0, 0)),
          pl.BlockSpec((_H, _C), lambda i: (0, 0)),
          pl.BlockSpec((1, _C), lambda i: (0, 0)),
      ],
      out_specs=pl.BlockSpec((_BLK, _C), lambda i: (i, 0)),
      out_shape=jax.ShapeDtypeStruct((_N, _C), jnp.float32),
  )(h, p, W2a, b2a.reshape(1, _H), W2b, b2b.reshape(1, _C))


def kernel(x, edge_index, W1a, b1a, W1b, b1b, W2a, b2a, W2b, b2b):
  src = edge_index[0].astype(jnp.int32)
  dst = edge_index[1].astype(jnp.int32)
  pad = _EP - _E
  srcp = jnp.concatenate([src, jnp.zeros((pad,), jnp.int32)])
  dstp = jnp.concatenate([dst, jnp.full((pad,), _N, jnp.int32)])
  zeros = jnp.zeros((_NP, _C), jnp.float32)

  idx2 = (srcp[None, :] * 2
          + jnp.arange(_NC, dtype=jnp.int32)[:, None]).reshape(_NC * _EP)
  dst2 = jnp.concatenate([dstp, dstp])

  p1 = _sc_segment_sum(x, srcp, dstp, zeros, _EP // _NC)[:, :_N]
  h = _mlp1(x, p1, W1a, b1a, W1b, b1b)
  p2 = _sc_segment_sum(h.reshape(2 * _N, _C), idx2, dst2, zeros,
                       _EP)[:, :_N]
  return _mlp2(h, p2, W2a, b2a, W2b, b2b)


# speedup vs baseline: 1.0737x; 1.0003x over previous
import functools

import jax
import jax.numpy as jnp
from jax import lax
from jax.experimental import pallas as pl
from jax.experimental.pallas import tpu as pltpu
from jax.experimental.pallas import tpu_sc as plsc

_N = 10000
_E = 320000
_C = 128
_H = 256

_NC = 2
_NS = 16
_B = 80
_EP = 327680
_NP = 10240
_RPS = _NP // _NS


def _sc_segment_sum(table, idxs, dsts, zeros, epc):
  eps = epc // _NS
  nb = eps // _B
  assert eps % _B == 0 and nb % 2 == 0

  mesh = plsc.VectorSubcoreMesh(core_axis_name="c", subcore_axis_name="s")

  @functools.partial(
      pl.kernel,
      out_type=jax.ShapeDtypeStruct((_NC, _NP, _C), jnp.float32),
      mesh=mesh,
      scratch_types=[
          pltpu.VMEM((_B,), jnp.int32),
          pltpu.VMEM((_B,), jnp.int32),
          pltpu.VMEM((_B, _C), jnp.float32),
          pltpu.VMEM_SHARED((_NP, _C), jnp.float32),
          pltpu.SemaphoreType.DMA,
      ],
  )
  def k(table_h, idxs_h, dsts_h, zero_h, out_h,
        idx0, dstb0, buf0, acc, sem0):
    c = lax.axis_index("c")
    s = lax.axis_index("s")

    pltpu.sync_copy(zero_h.at[pl.ds(s * _RPS, _RPS)],
                    acc.at[pl.ds(s * _RPS, _RPS)])
    plsc.subcore_barrier()

    base = c * epc + s * eps

    def body(i, carry):
      off = pl.multiple_of(base + i * _B, 8)
      pltpu.sync_copy(idxs_h.at[pl.ds(off, _B)], idx0)
      pltpu.async_copy(table_h.at[idx0], buf0, sem0).wait()
      pltpu.sync_copy(dsts_h.at[pl.ds(off, _B)], dstb0)
      pltpu.sync_copy(buf0, acc.at[dstb0], add=True)
      return carry

    lax.fori_loop(0, nb, body, 0)
    plsc.subcore_barrier()
    pltpu.sync_copy(acc.at[pl.ds(s * _RPS, _RPS)],
                    out_h.at[c, pl.ds(s * _RPS, _RPS)])

  return k(table, idxs, dsts, zeros)


_BLK = 400


def _mlp1_body(x_ref, p_ref, wa_ref, ba_ref, wb_ref, bb_ref, h_ref):
  t = x_ref[...] + p_ref[0] + p_ref[1]
  a = jnp.maximum(
      jnp.dot(t, wa_ref[...], preferred_element_type=jnp.float32)
      + ba_ref[...], 0.0)
  h = jnp.maximum(
      jnp.dot(a, wb_ref[...], preferred_element_type=jnp.float32)
      + bb_ref[...], 0.0)
  h_ref[...] = h


def _mlp1(x, p, W1a, b1a, W1b, b1b):
  return pl.pallas_call(
      _mlp1_body,
      grid=(_N // _BLK,),
      in_specs=[
          pl.BlockSpec((_BLK, _C), lambda i: (i, 0)),
          pl.BlockSpec((_NC, _BLK, _C), lambda i: (0, i, 0)),
          pl.BlockSpec((_C, _H), lambda i: (0, 0)),
          pl.BlockSpec((1, _H), lambda i: (0, 0)),
          pl.BlockSpec((_H, _H), lambda i: (0, 0)),
          pl.BlockSpec((1, _H), lambda i: (0, 0)),
      ],
      out_specs=pl.BlockSpec((_BLK, _H), lambda i: (i, 0)),
      out_shape=jax.ShapeDtypeStruct((_N, _H), jnp.float32),
  )(x, p, W1a, b1a.reshape(1, _H), W1b, b1b.reshape(1, _H))


def _mlp2_body(h_ref, p_ref, wa_ref, ba_ref, wb_ref, bb_ref, o_ref):
  t = h_ref[...] + jnp.concatenate([p_ref[0], p_ref[1]], axis=1)
  z = jnp.maximum(
      jnp.dot(t, wa_ref[...], preferred_element_type=jnp.float32)
      + ba_ref[...], 0.0)
  u = jnp.dot(z, wb_ref[...], preferred_element_type=jnp.float32) + bb_ref[...]
  o_ref[...] = 1.0 / (1.0 + jnp.exp(-u))


def _mlp2(h, p, W2a, b2a, W2b, b2b):
  return pl.pallas_call(
      _mlp2_body,
      grid=(_N // _BLK,),
      in_specs=[
          pl.BlockSpec((_BLK, _H), lambda i: (i, 0)),
          pl.BlockSpec((_NC, _BLK, _C), lambda i: (0, i, 0)),
          pl.BlockSpec((_H, _H), lambda i: (0, 0)),
          pl.BlockSpec((1, _H), lambda i: (0, 0)),
          pl.BlockSpec((_H, _C), lambda i: (0, 0)),
          pl.BlockSpec((1, _C), lambda i: (0, 0)),
      ],
      out_specs=pl.BlockSpec((_BLK, _C), lambda i: (i, 0)),
      out_shape=jax.ShapeDtypeStruct((_N, _C), jnp.float32),
  )(h, p, W2a, b2a.reshape(1, _H), W2b, b2b.reshape(1, _C))


def kernel(x, edge_index, W1a, b1a, W1b, b1b, W2a, b2a, W2b, b2b):
  src = edge_index[0].astype(jnp.int32)
  dst = edge_index[1].astype(jnp.int32)
  pad = _EP - _E
  srcp = jnp.concatenate([src, jnp.zeros((pad,), jnp.int32)])
  dstp = jnp.concatenate(
      [dst, _N + (jnp.arange(pad, dtype=jnp.int32) % (_NP - _N))])
  zeros = jnp.zeros((_NP, _C), jnp.float32)

  idx2 = (srcp[None, :] * 2
          + jnp.arange(_NC, dtype=jnp.int32)[:, None]).reshape(_NC * _EP)
  dst2 = jnp.concatenate([dstp, dstp])

  p1 = _sc_segment_sum(x, srcp, dstp, zeros, _EP // _NC)[:, :_N]
  h = _mlp1(x, p1, W1a, b1a, W1b, b1b)
  p2 = _sc_segment_sum(h.reshape(2 * _N, _C), idx2, dst2, zeros,
                       _EP)[:, :_N]
  return _mlp2(h, p2, W2a, b2a, W2b, b2b)


# speedup vs baseline: 1.7321x; 1.6132x over previous
import functools

import jax
import jax.numpy as jnp
from jax import lax
from jax.experimental import pallas as pl
from jax.experimental.pallas import tpu as pltpu
from jax.experimental.pallas import tpu_sc as plsc

_N = 10000
_E = 320000
_C = 128
_H = 256

_NC = 2
_NS = 16
_B = 80
_EP = 327680
_NP = 10240
_RPS = _NP // _NS


def _sc_segment_sum(table, idxs, dsts, zeros, epc):
  eps = epc // _NS
  nb = eps // _B
  assert eps % _B == 0 and nb % 2 == 0

  mesh = plsc.VectorSubcoreMesh(core_axis_name="c", subcore_axis_name="s")

  @functools.partial(
      pl.kernel,
      out_type=jax.ShapeDtypeStruct((_NC, _NP, _C), jnp.float32),
      mesh=mesh,
      scratch_types=[
          pltpu.VMEM((_B,), jnp.int32),
          pltpu.VMEM((_B,), jnp.int32),
          pltpu.VMEM((_B, _C), jnp.float32),
          pltpu.VMEM_SHARED((_NP, _C), jnp.float32),
          pltpu.SemaphoreType.DMA,
      ],
  )
  def k(table_h, idxs_h, dsts_h, zero_h, out_h,
        idx0, dstb0, buf0, acc, sem0):
    c = lax.axis_index("c")
    s = lax.axis_index("s")

    pltpu.sync_copy(zero_h.at[pl.ds(s * _RPS, _RPS)],
                    acc.at[pl.ds(s * _RPS, _RPS)])
    plsc.subcore_barrier()

    base = c * epc + s * eps

    def body(i, carry):
      off = pl.multiple_of(base + i * _B, 8)
      pltpu.sync_copy(idxs_h.at[pl.ds(off, _B)], idx0)
      pltpu.async_copy(table_h.at[idx0], buf0, sem0).wait()
      pltpu.sync_copy(dsts_h.at[pl.ds(off, _B)], dstb0)
      pltpu.sync_copy(buf0, acc.at[dstb0], add=True)
      return carry

    lax.fori_loop(0, nb, body, 0)
    plsc.subcore_barrier()
    pltpu.sync_copy(acc.at[pl.ds(s * _RPS, _RPS)],
                    out_h.at[c, pl.ds(s * _RPS, _RPS)])

  return k(table, idxs, dsts, zeros)


_BLK = 400


def _mlp1_body(x_ref, p_ref, wa_ref, ba_ref, wb_ref, bb_ref, h_ref):
  t = x_ref[...] + p_ref[0] + p_ref[1]
  a = jnp.maximum(
      jnp.dot(t, wa_ref[...], preferred_element_type=jnp.float32)
      + ba_ref[...], 0.0)
  h = jnp.maximum(
      jnp.dot(a, wb_ref[...], preferred_element_type=jnp.float32)
      + bb_ref[...], 0.0)
  h_ref[...] = h


def _mlp1(x, p, W1a, b1a, W1b, b1b):
  return pl.pallas_call(
      _mlp1_body,
      grid=(_N // _BLK,),
      in_specs=[
          pl.BlockSpec((_BLK, _C), lambda i: (i, 0)),
          pl.BlockSpec((_NC, _BLK, _C), lambda i: (0, i, 0)),
          pl.BlockSpec((_C, _H), lambda i: (0, 0)),
          pl.BlockSpec((1, _H), lambda i: (0, 0)),
          pl.BlockSpec((_H, _H), lambda i: (0, 0)),
          pl.BlockSpec((1, _H), lambda i: (0, 0)),
      ],
      out_specs=pl.BlockSpec((_BLK, _H), lambda i: (i, 0)),
      out_shape=jax.ShapeDtypeStruct((_N, _H), jnp.float32),
  )(x, p, W1a, b1a.reshape(1, _H), W1b, b1b.reshape(1, _H))


def _mlp2_body(h_ref, p_ref, wa_ref, ba_ref, wb_ref, bb_ref, o_ref):
  t = h_ref[...] + jnp.concatenate([p_ref[0], p_ref[1]], axis=1)
  z = jnp.maximum(
      jnp.dot(t, wa_ref[...], preferred_element_type=jnp.float32)
      + ba_ref[...], 0.0)
  u = jnp.dot(z, wb_ref[...], preferred_element_type=jnp.float32) + bb_ref[...]
  o_ref[...] = 1.0 / (1.0 + jnp.exp(-u))


def _mlp2(h, p, W2a, b2a, W2b, b2b):
  return pl.pallas_call(
      _mlp2_body,
      grid=(_N // _BLK,),
      in_specs=[
          pl.BlockSpec((_BLK, _H), lambda i: (i, 0)),
          pl.BlockSpec((_NC, _BLK, _C), lambda i: (0, i, 0)),
          pl.BlockSpec((_H, _H), lambda i: (0, 0)),
          pl.BlockSpec((1, _H), lambda i: (0, 0)),
          pl.BlockSpec((_H, _C), lambda i: (0, 0)),
          pl.BlockSpec((1, _C), lambda i: (0, 0)),
      ],
      out_specs=pl.BlockSpec((_BLK, _C), lambda i: (i, 0)),
      out_shape=jax.ShapeDtypeStruct((_N, _C), jnp.float32),
  )(h, p, W2a, b2a.reshape(1, _H), W2b, b2b.reshape(1, _C))


def kernel(x, edge_index, W1a, b1a, W1b, b1b, W2a, b2a, W2b, b2b):
  src = edge_index[0].astype(jnp.int32)
  dst = edge_index[1].astype(jnp.int32)
  pad = _EP - _E
  ar = jnp.arange(pad, dtype=jnp.int32)
  srcp = jnp.concatenate([src, ar % _N])
  dstp = jnp.concatenate([dst, _N + (ar % (_NP - _N))])
  zeros = jnp.zeros((_NP, _C), jnp.float32)

  idx2 = (srcp[None, :] * 2
          + jnp.arange(_NC, dtype=jnp.int32)[:, None]).reshape(_NC * _EP)
  dst2 = jnp.concatenate([dstp, dstp])

  p1 = _sc_segment_sum(x, srcp, dstp, zeros, _EP // _NC)[:, :_N]
  h = _mlp1(x, p1, W1a, b1a, W1b, b1b)
  p2 = _sc_segment_sum(h.reshape(2 * _N, _C), idx2, dst2, zeros,
                       _EP)[:, :_N]
  return _mlp2(h, p2, W2a, b2a, W2b, b2b)


# speedup vs baseline: 2.7751x; 1.6021x over previous
import functools

import jax
import jax.numpy as jnp
from jax import lax
from jax.experimental import pallas as pl
from jax.experimental.pallas import tpu as pltpu
from jax.experimental.pallas import tpu_sc as plsc

_N = 10000
_E = 320000
_C = 128
_H = 256

_NC = 2
_NS = 16
_B = 80
_EP = 327680
_NP = 10240
_RPS = _NP // _NS


def _sc_segment_sum(table, idxs, dsts, zeros, epc):
  eps = epc // _NS
  nb = eps // _B
  assert eps % _B == 0 and nb % 2 == 0

  mesh = plsc.VectorSubcoreMesh(core_axis_name="c", subcore_axis_name="s")

  @functools.partial(
      pl.kernel,
      out_type=jax.ShapeDtypeStruct((_NC, _NP, _C), jnp.float32),
      mesh=mesh,
      scratch_types=[
          pltpu.VMEM((_B,), jnp.int32),
          pltpu.VMEM((_B,), jnp.int32),
          pltpu.VMEM((_B,), jnp.int32),
          pltpu.VMEM((_B,), jnp.int32),
          pltpu.VMEM((_B, _C), jnp.float32),
          pltpu.VMEM((_B, _C), jnp.float32),
          pltpu.VMEM_SHARED((_NP, _C), jnp.float32),
          pltpu.SemaphoreType.DMA,
          pltpu.SemaphoreType.DMA,
      ],
  )
  def k(table_h, idxs_h, dsts_h, zero_h, out_h,
        idx0, idx1, dstb0, dstb1, buf0, buf1, acc, sem0, sem1):
    c = lax.axis_index("c")
    s = lax.axis_index("s")

    pltpu.sync_copy(zero_h.at[pl.ds(s * _RPS, _RPS)],
                    acc.at[pl.ds(s * _RPS, _RPS)])
    plsc.subcore_barrier()

    base = c * epc + s * eps

    def body(i, carry):
      off0 = pl.multiple_of(base + (2 * i) * _B, 8)
      off1 = pl.multiple_of(base + (2 * i + 1) * _B, 8)
      pltpu.sync_copy(idxs_h.at[pl.ds(off0, _B)], idx0)
      cp0 = pltpu.async_copy(table_h.at[idx0], buf0, sem0)
      pltpu.sync_copy(idxs_h.at[pl.ds(off1, _B)], idx1)
      cp1 = pltpu.async_copy(table_h.at[idx1], buf1, sem1)
      pltpu.sync_copy(dsts_h.at[pl.ds(off0, _B)], dstb0)
      pltpu.sync_copy(dsts_h.at[pl.ds(off1, _B)], dstb1)
      cp0.wait()
      pltpu.sync_copy(buf0, acc.at[dstb0], add=True)
      cp1.wait()
      pltpu.sync_copy(buf1, acc.at[dstb1], add=True)
      return carry

    lax.fori_loop(0, nb // 2, body, 0)
    plsc.subcore_barrier()
    pltpu.sync_copy(acc.at[pl.ds(s * _RPS, _RPS)],
                    out_h.at[c, pl.ds(s * _RPS, _RPS)])

  return k(table, idxs, dsts, zeros)


_BLK = 400


def _mlp1_body(x_ref, p_ref, wa_ref, ba_ref, wb_ref, bb_ref, h_ref):
  t = x_ref[...] + p_ref[0] + p_ref[1]
  a = jnp.maximum(
      jnp.dot(t, wa_ref[...], preferred_element_type=jnp.float32)
      + ba_ref[...], 0.0)
  h = jnp.maximum(
      jnp.dot(a, wb_ref[...], preferred_element_type=jnp.float32)
      + bb_ref[...], 0.0)
  h_ref[...] = h


def _mlp1(x, p, W1a, b1a, W1b, b1b):
  return pl.pallas_call(
      _mlp1_body,
      grid=(_N // _BLK,),
      in_specs=[
          pl.BlockSpec((_BLK, _C), lambda i: (i, 0)),
          pl.BlockSpec((_NC, _BLK, _C), lambda i: (0, i, 0)),
          pl.BlockSpec((_C, _H), lambda i: (0, 0)),
          pl.BlockSpec((1, _H), lambda i: (0, 0)),
          pl.BlockSpec((_H, _H), lambda i: (0, 0)),
          pl.BlockSpec((1, _H), lambda i: (0, 0)),
      ],
      out_specs=pl.BlockSpec((_BLK, _H), lambda i: (i, 0)),
      out_shape=jax.ShapeDtypeStruct((_N, _H), jnp.float32),
  )(x, p, W1a, b1a.reshape(1, _H), W1b, b1b.reshape(1, _H))


def _mlp2_body(h_ref, p_ref, wa_ref, ba_ref, wb_ref, bb_ref, o_ref):
  t = h_ref[...] + jnp.concatenate([p_ref[0], p_ref[1]], axis=1)
  z = jnp.maximum(
      jnp.dot(t, wa_ref[...], preferred_element_type=jnp.float32)
      + ba_ref[...], 0.0)
  u = jnp.dot(z, wb_ref[...], preferred_element_type=jnp.float32) + bb_ref[...]
  o_ref[...] = 1.0 / (1.0 + jnp.exp(-u))


def _mlp2(h, p, W2a, b2a, W2b, b2b):
  return pl.pallas_call(
      _mlp2_body,
      grid=(_N // _BLK,),
      in_specs=[
          pl.BlockSpec((_BLK, _H), lambda i: (i, 0)),
          pl.BlockSpec((_NC, _BLK, _C), lambda i: (0, i, 0)),
          pl.BlockSpec((_H, _H), lambda i: (0, 0)),
          pl.BlockSpec((1, _H), lambda i: (0, 0)),
          pl.BlockSpec((_H, _C), lambda i: (0, 0)),
          pl.BlockSpec((1, _C), lambda i: (0, 0)),
      ],
      out_specs=pl.BlockSpec((_BLK, _C), lambda i: (i, 0)),
      out_shape=jax.ShapeDtypeStruct((_N, _C), jnp.float32),
  )(h, p, W2a, b2a.reshape(1, _H), W2b, b2b.reshape(1, _C))


def kernel(x, edge_index, W1a, b1a, W1b, b1b, W2a, b2a, W2b, b2b):
  src = edge_index[0].astype(jnp.int32)
  dst = edge_index[1].astype(jnp.int32)
  pad = _EP - _E
  ar = jnp.arange(pad, dtype=jnp.int32)
  srcp = jnp.concatenate([src, ar % _N])
  dstp = jnp.concatenate([dst, _N + (ar % (_NP - _N))])
  zeros = jnp.zeros((_NP, _C), jnp.float32)

  idx2 = (srcp[None, :] * 2
          + jnp.arange(_NC, dtype=jnp.int32)[:, None]).reshape(_NC * _EP)
  dst2 = jnp.concatenate([dstp, dstp])

  p1 = _sc_segment_sum(x, srcp, dstp, zeros, _EP // _NC)[:, :_N]
  h = _mlp1(x, p1, W1a, b1a, W1b, b1b)
  p2 = _sc_segment_sum(h.reshape(2 * _N, _C), idx2, dst2, zeros,
                       _EP)[:, :_N]
  return _mlp2(h, p2, W2a, b2a, W2b, b2b)


# speedup vs baseline: 3.1675x; 1.1414x over previous
import functools

import jax
import jax.numpy as jnp
from jax import lax
from jax.experimental import pallas as pl
from jax.experimental.pallas import tpu as pltpu
from jax.experimental.pallas import tpu_sc as plsc

_N = 10000
_E = 320000
_C = 128
_H = 256

_NC = 2
_NS = 16
_B = 128
_EP = 327680
_NP = 10240
_RPS = _NP // _NS


def _sc_segment_sum(table, idxs, dsts, zeros, epc):
  eps = epc // _NS
  nb = eps // _B
  assert eps % _B == 0 and nb % 2 == 0

  mesh = plsc.VectorSubcoreMesh(core_axis_name="c", subcore_axis_name="s")

  @functools.partial(
      pl.kernel,
      out_type=jax.ShapeDtypeStruct((_NC, _NP, _C), jnp.float32),
      mesh=mesh,
      scratch_types=[
          pltpu.VMEM((_B,), jnp.int32),
          pltpu.VMEM((_B,), jnp.int32),
          pltpu.VMEM((_B,), jnp.int32),
          pltpu.VMEM((_B,), jnp.int32),
          pltpu.VMEM((_B, _C), jnp.float32),
          pltpu.VMEM((_B, _C), jnp.float32),
          pltpu.VMEM_SHARED((_NP, _C), jnp.float32),
          pltpu.SemaphoreType.DMA,
          pltpu.SemaphoreType.DMA,
      ],
  )
  def k(table_h, idxs_h, dsts_h, zero_h, out_h,
        idx0, idx1, dstb0, dstb1, buf0, buf1, acc, sem0, sem1):
    c = lax.axis_index("c")
    s = lax.axis_index("s")

    pltpu.sync_copy(zero_h.at[pl.ds(s * _RPS, _RPS)],
                    acc.at[pl.ds(s * _RPS, _RPS)])
    plsc.subcore_barrier()

    base = c * epc + s * eps

    def body(i, carry):
      off0 = pl.multiple_of(base + (2 * i) * _B, 8)
      off1 = pl.multiple_of(base + (2 * i + 1) * _B, 8)
      pltpu.sync_copy(idxs_h.at[pl.ds(off0, _B)], idx0)
      cp0 = pltpu.async_copy(table_h.at[idx0], buf0, sem0)
      pltpu.sync_copy(idxs_h.at[pl.ds(off1, _B)], idx1)
      cp1 = pltpu.async_copy(table_h.at[idx1], buf1, sem1)
      pltpu.sync_copy(dsts_h.at[pl.ds(off0, _B)], dstb0)
      pltpu.sync_copy(dsts_h.at[pl.ds(off1, _B)], dstb1)
      cp0.wait()
      pltpu.sync_copy(buf0, acc.at[dstb0], add=True)
      cp1.wait()
      pltpu.sync_copy(buf1, acc.at[dstb1], add=True)
      return carry

    lax.fori_loop(0, nb // 2, body, 0)
    plsc.subcore_barrier()
    pltpu.sync_copy(acc.at[pl.ds(s * _RPS, _RPS)],
                    out_h.at[c, pl.ds(s * _RPS, _RPS)])

  return k(table, idxs, dsts, zeros)


_BLK = 400


def _mlp1_body(x_ref, p_ref, wa_ref, ba_ref, wb_ref, bb_ref, h_ref):
  t = x_ref[...] + p_ref[0] + p_ref[1]
  a = jnp.maximum(
      jnp.dot(t, wa_ref[...], preferred_element_type=jnp.float32)
      + ba_ref[...], 0.0)
  h = jnp.maximum(
      jnp.dot(a, wb_ref[...], preferred_element_type=jnp.float32)
      + bb_ref[...], 0.0)
  h_ref[...] = h


def _mlp1(x, p, W1a, b1a, W1b, b1b):
  return pl.pallas_call(
      _mlp1_body,
      grid=(_N // _BLK,),
      in_specs=[
          pl.BlockSpec((_BLK, _C), lambda i: (i, 0)),
          pl.BlockSpec((_NC, _BLK, _C), lambda i: (0, i, 0)),
          pl.BlockSpec((_C, _H), lambda i: (0, 0)),
          pl.BlockSpec((1, _H), lambda i: (0, 0)),
          pl.BlockSpec((_H, _H), lambda i: (0, 0)),
          pl.BlockSpec((1, _H), lambda i: (0, 0)),
      ],
      out_specs=pl.BlockSpec((_BLK, _H), lambda i: (i, 0)),
      out_shape=jax.ShapeDtypeStruct((_N, _H), jnp.float32),
  )(x, p, W1a, b1a.reshape(1, _H), W1b, b1b.reshape(1, _H))


def _mlp2_body(h_ref, p_ref, wa_ref, ba_ref, wb_ref, bb_ref, o_ref):
  t = h_ref[...] + jnp.concatenate([p_ref[0], p_ref[1]], axis=1)
  z = jnp.maximum(
      jnp.dot(t, wa_ref[...], preferred_element_type=jnp.float32)
      + ba_ref[...], 0.0)
  u = jnp.dot(z, wb_ref[...], preferred_element_type=jnp.float32) + bb_ref[...]
  o_ref[...] = 1.0 / (1.0 + jnp.exp(-u))


def _mlp2(h, p, W2a, b2a, W2b, b2b):
  return pl.pallas_call(
      _mlp2_body,
      grid=(_N // _BLK,),
      in_specs=[
          pl.BlockSpec((_BLK, _H), lambda i: (i, 0)),
          pl.BlockSpec((_NC, _BLK, _C), lambda i: (0, i, 0)),
          pl.BlockSpec((_H, _H), lambda i: (0, 0)),
          pl.BlockSpec((1, _H), lambda i: (0, 0)),
          pl.BlockSpec((_H, _C), lambda i: (0, 0)),
          pl.BlockSpec((1, _C), lambda i: (0, 0)),
      ],
      out_specs=pl.BlockSpec((_BLK, _C), lambda i: (i, 0)),
      out_shape=jax.ShapeDtypeStruct((_N, _C), jnp.float32),
  )(h, p, W2a, b2a.reshape(1, _H), W2b, b2b.reshape(1, _C))


def kernel(x, edge_index, W1a, b1a, W1b, b1b, W2a, b2a, W2b, b2b):
  src = edge_index[0].astype(jnp.int32)
  dst = edge_index[1].astype(jnp.int32)
  pad = _EP - _E
  ar = jnp.arange(pad, dtype=jnp.int32)
  srcp = jnp.concatenate([src, ar % _N])
  dstp = jnp.concatenate([dst, _N + (ar % (_NP - _N))])
  zeros = jnp.zeros((_NP, _C), jnp.float32)

  idx2 = (srcp[None, :] * 2
          + jnp.arange(_NC, dtype=jnp.int32)[:, None]).reshape(_NC * _EP)
  dst2 = jnp.concatenate([dstp, dstp])

  p1 = _sc_segment_sum(x, srcp, dstp, zeros, _EP // _NC)[:, :_N]
  h = _mlp1(x, p1, W1a, b1a, W1b, b1b)
  p2 = _sc_segment_sum(h.reshape(2 * _N, _C), idx2, dst2, zeros,
                       _EP)[:, :_N]
  return _mlp2(h, p2, W2a, b2a, W2b, b2b)


# speedup vs baseline: 3.3135x; 1.0461x over previous
import functools

import jax
import jax.numpy as jnp
from jax import lax
from jax.experimental import pallas as pl
from jax.experimental.pallas import tpu as pltpu
from jax.experimental.pallas import tpu_sc as plsc

_N = 10000
_E = 320000
_C = 128
_H = 256

_NC = 2
_NS = 16
_B = 128
_EP = 327680
_NP = 10240
_RPS = _NP // _NS


def _sc_segment_sum(table, idxs, dsts, zeros, epc):
  eps = epc // _NS
  nb = eps // _B
  assert eps % _B == 0 and nb % 2 == 0

  mesh = plsc.VectorSubcoreMesh(core_axis_name="c", subcore_axis_name="s")

  @functools.partial(
      pl.kernel,
      out_type=jax.ShapeDtypeStruct((_NC, _NP, _C), jnp.float32),
      mesh=mesh,
      scratch_types=[
          pltpu.VMEM((_B,), jnp.int32),
          pltpu.VMEM((_B,), jnp.int32),
          pltpu.VMEM((_B,), jnp.int32),
          pltpu.VMEM((_B,), jnp.int32),
          pltpu.VMEM((_B, _C), jnp.float32),
          pltpu.VMEM((_B, _C), jnp.float32),
          pltpu.VMEM_SHARED((_NP, _C), jnp.float32),
          pltpu.SemaphoreType.DMA,
          pltpu.SemaphoreType.DMA,
          pltpu.SemaphoreType.DMA,
          pltpu.SemaphoreType.DMA,
      ],
  )
  def k(table_h, idxs_h, dsts_h, zero_h, out_h,
        idx0, idx1, dstb0, dstb1, buf0, buf1, acc, sem0, sem1, sem2, sem3):
    c = lax.axis_index("c")
    s = lax.axis_index("s")

    pltpu.sync_copy(zero_h.at[pl.ds(s * _RPS, _RPS)],
                    acc.at[pl.ds(s * _RPS, _RPS)])
    plsc.subcore_barrier()

    base = c * epc + s * eps

    lanes = lax.iota(jnp.int32, 16)
    for j in range(_B // 16):
      dstb0[pl.ds(j * 16, 16)] = lanes + (_N + j * 16)
      dstb1[pl.ds(j * 16, 16)] = lanes + (_N + j * 16)
    pltpu.async_copy(buf0, acc.at[dstb0], sem2, add=True)
    pltpu.async_copy(buf1, acc.at[dstb1], sem3, add=True)

    def body(i, carry):
      off0 = pl.multiple_of(base + (2 * i) * _B, 8)
      off1 = pl.multiple_of(base + (2 * i + 1) * _B, 8)
      pltpu.make_async_copy(buf0, acc.at[dstb0], sem2).wait()
      pltpu.sync_copy(idxs_h.at[pl.ds(off0, _B)], idx0)
      cp0 = pltpu.async_copy(table_h.at[idx0], buf0, sem0)
      pltpu.make_async_copy(buf1, acc.at[dstb1], sem3).wait()
      pltpu.sync_copy(idxs_h.at[pl.ds(off1, _B)], idx1)
      cp1 = pltpu.async_copy(table_h.at[idx1], buf1, sem1)
      pltpu.sync_copy(dsts_h.at[pl.ds(off0, _B)], dstb0)
      pltpu.sync_copy(dsts_h.at[pl.ds(off1, _B)], dstb1)
      cp0.wait()
      pltpu.async_copy(buf0, acc.at[dstb0], sem2, add=True)
      cp1.wait()
      pltpu.async_copy(buf1, acc.at[dstb1], sem3, add=True)
      return carry

    lax.fori_loop(0, nb // 2, body, 0)
    pltpu.make_async_copy(buf0, acc.at[dstb0], sem2).wait()
    pltpu.make_async_copy(buf1, acc.at[dstb1], sem3).wait()
    plsc.subcore_barrier()
    pltpu.sync_copy(acc.at[pl.ds(s * _RPS, _RPS)],
                    out_h.at[c, pl.ds(s * _RPS, _RPS)])

  return k(table, idxs, dsts, zeros)


_BLK = 400


def _mlp1_body(x_ref, p_ref, wa_ref, ba_ref, wb_ref, bb_ref, h_ref):
  t = x_ref[...] + p_ref[0] + p_ref[1]
  a = jnp.maximum(
      jnp.dot(t, wa_ref[...], preferred_element_type=jnp.float32)
      + ba_ref[...], 0.0)
  h = jnp.maximum(
      jnp.dot(a, wb_ref[...], preferred_element_type=jnp.float32)
      + bb_ref[...], 0.0)
  h_ref[...] = h


def _mlp1(x, p, W1a, b1a, W1b, b1b):
  return pl.pallas_call(
      _mlp1_body,
      grid=(_N // _BLK,),
      in_specs=[
          pl.BlockSpec((_BLK, _C), lambda i: (i, 0)),
          pl.BlockSpec((_NC, _BLK, _C), lambda i: (0, i, 0)),
          pl.BlockSpec((_C, _H), lambda i: (0, 0)),
          pl.BlockSpec((1, _H), lambda i: (0, 0)),
          pl.BlockSpec((_H, _H), lambda i: (0, 0)),
          pl.BlockSpec((1, _H), lambda i: (0, 0)),
      ],
      out_specs=pl.BlockSpec((_BLK, _H), lambda i: (i, 0)),
      out_shape=jax.ShapeDtypeStruct((_N, _H), jnp.float32),
  )(x, p, W1a, b1a.reshape(1, _H), W1b, b1b.reshape(1, _H))


def _mlp2_body(h_ref, p_ref, wa_ref, ba_ref, wb_ref, bb_ref, o_ref):
  t = h_ref[...] + jnp.concatenate([p_ref[0], p_ref[1]], axis=1)
  z = jnp.maximum(
      jnp.dot(t, wa_ref[...], preferred_element_type=jnp.float32)
      + ba_ref[...], 0.0)
  u = jnp.dot(z, wb_ref[...], preferred_element_type=jnp.float32) + bb_ref[...]
  o_ref[...] = 1.0 / (1.0 + jnp.exp(-u))


def _mlp2(h, p, W2a, b2a, W2b, b2b):
  return pl.pallas_call(
      _mlp2_body,
      grid=(_N // _BLK,),
      in_specs=[
          pl.BlockSpec((_BLK, _H), lambda i: (i, 0)),
          pl.BlockSpec((_NC, _BLK, _C), lambda i: (0, i, 0)),
          pl.BlockSpec((_H, _H), lambda i: (0, 0)),
          pl.BlockSpec((1, _H), lambda i: (0, 0)),
          pl.BlockSpec((_H, _C), lambda i: (0, 0)),
          pl.BlockSpec((1, _C), lambda i: (0, 0)),
      ],
      out_specs=pl.BlockSpec((_BLK, _C), lambda i: (i, 0)),
      out_shape=jax.ShapeDtypeStruct((_N, _C), jnp.float32),
  )(h, p, W2a, b2a.reshape(1, _H), W2b, b2b.reshape(1, _C))


def kernel(x, edge_index, W1a, b1a, W1b, b1b, W2a, b2a, W2b, b2b):
  src = edge_index[0].astype(jnp.int32)
  dst = edge_index[1].astype(jnp.int32)
  pad = _EP - _E
  ar = jnp.arange(pad, dtype=jnp.int32)
  srcp = jnp.concatenate([src, ar % _N])
  dstp = jnp.concatenate([dst, _N + (ar % (_NP - _N))])
  zeros = jnp.zeros((_NP, _C), jnp.float32)

  idx2 = (srcp[None, :] * 2
          + jnp.arange(_NC, dtype=jnp.int32)[:, None]).reshape(_NC * _EP)
  dst2 = jnp.concatenate([dstp, dstp])

  p1 = _sc_segment_sum(x, srcp, dstp, zeros, _EP // _NC)[:, :_N]
  h = _mlp1(x, p1, W1a, b1a, W1b, b1b)
  p2 = _sc_segment_sum(h.reshape(2 * _N, _C), idx2, dst2, zeros,
                       _EP)[:, :_N]
  return _mlp2(h, p2, W2a, b2a, W2b, b2b)
